# Initial kernel scaffold; baseline (speedup 1.0000x reference)
import jax
import jax.numpy as jnp
from jax.experimental import pallas as pl


def kernel(edge_index, h, e, emb, Ws, bs, gammas, betas, W1, b1, W2, b2, W3, b3):
    raise NotImplementedError("write your pallas kernel here")


# speedup vs baseline: 4.9336x; 4.9336x over previous
import functools

import jax
import jax.numpy as jnp
from jax import lax
from jax.experimental import pallas as pl
from jax.experimental.pallas import tpu as pltpu
from jax.experimental.pallas import tpu_sc as plsc

NN = 10000
EE = 320000
HH = 128
LL = 4
NROWS = 10240
DUMMY = NN
NC = 2
NT = 16
CH = 128
CPT = 79
EPT = CPT * CH
ROWS_PT = NROWS // NT

_sc_mesh = plsc.VectorSubcoreMesh(core_axis_name="c", subcore_axis_name="s")


@functools.partial(
    pl.kernel,
    out_type=jax.ShapeDtypeStruct((NC, NROWS), jnp.float32),
    mesh=_sc_mesh,
    scratch_types=[
        pltpu.VMEM((CPT, CH), jnp.int32),
        pltpu.VMEM((CH,), jnp.float32),
        pltpu.VMEM((NROWS,), jnp.float32),
        pltpu.VMEM_SHARED((NROWS,), jnp.float32),
    ],
)
def _sc_degrees(srcp_hbm, dstp_hbm, out_hbm, idxv, onesv, zb, deg):
    c = lax.axis_index("c")
    s = lax.axis_index("s")
    one16 = jnp.full((16,), 1.0, jnp.float32)
    for k in range(CH // 16):
        onesv[pl.ds(k * 16, 16)] = one16

    @pl.when(s == 0)
    def _zero():
        zero16 = jnp.zeros((16,), jnp.float32)

        def zbody(i, carry):
            zb[pl.ds(i * 16, 16)] = zero16
            return carry

        lax.fori_loop(0, NROWS // 16, zbody, 0)
        pltpu.sync_copy(zb, deg)

    plsc.subcore_barrier()

    def hist(idx_hbm):
        for half in range(NC):
            pltpu.sync_copy(idx_hbm.at[half * NT + s], idxv)

            def body(j, carry):
                pltpu.sync_copy(onesv, deg.at[idxv.at[j]], add=True)
                return carry

            lax.fori_loop(0, CPT, body, 0)

    @pl.when(c == 0)
    def _src():
        hist(srcp_hbm)

    @pl.when(c == 1)
    def _dst():
        hist(dstp_hbm)

    plsc.subcore_barrier()

    @pl.when(s == 0)
    def _out():
        pltpu.sync_copy(deg, out_hbm.at[c])


@functools.partial(
    pl.kernel,
    out_type=jax.ShapeDtypeStruct((NC, NROWS, HH), jnp.float32),
    mesh=_sc_mesh,
    scratch_types=[
        pltpu.VMEM((CPT, CH), jnp.int32),
        pltpu.VMEM((CPT, CH), jnp.int32),
        pltpu.VMEM((CH, HH), jnp.float32),
        pltpu.VMEM_SHARED((NROWS, HH), jnp.float32),
        pltpu.SemaphoreType.DMA,
    ],
)
def _sc_spmm(xs_hbm, srcp_hbm, dstp_hbm, out_hbm, sidx, didx, gbuf, agg, sem):
    c = lax.axis_index("c")
    s = lax.axis_index("s")
    w = c * NT + s

    zero16 = jnp.zeros((16,), jnp.float32)

    def zbody(i, carry):
        r = i // (HH // 16)
        k = (i % (HH // 16)) * 16
        gbuf[r, pl.ds(k, 16)] = zero16
        return carry

    lax.fori_loop(0, CH * (HH // 16), zbody, 0)

    base = s * ROWS_PT
    for k in range(ROWS_PT // CH):
        pltpu.sync_copy(gbuf, agg.at[pl.ds(base + k * CH, CH), :])

    pltpu.sync_copy(srcp_hbm.at[w], sidx)
    pltpu.sync_copy(dstp_hbm.at[w], didx)
    plsc.subcore_barrier()

    def body(j, carry):
        pltpu.async_copy(xs_hbm.at[sidx.at[j]], gbuf, sem).wait()
        pltpu.sync_copy(gbuf, agg.at[didx.at[j]], add=True)
        return carry

    lax.fori_loop(0, CPT, body, 0)

    plsc.subcore_barrier()
    pltpu.sync_copy(agg.at[pl.ds(base, ROWS_PT), :],
                    out_hbm.at[c, pl.ds(base, ROWS_PT), :])


def _tc_prep_body(h_ref, emb_ref, degT_ref, x0_ref, xs_ref):
    hv = h_ref[...]
    oh = (hv == lax.broadcasted_iota(jnp.int32, (NN, 28), 1)).astype(jnp.float32)
    x0 = lax.dot_general(oh, emb_ref[...], (((1,), (0,)), ((), ())),
                         preferred_element_type=jnp.float32)
    norm = lax.rsqrt(jnp.maximum(degT_ref[...], 1.0))
    x0_ref[...] = x0
    xs_ref[0:NN, :] = x0 * norm[:, 0:1]
    xs_ref[NN:NROWS, :] = jnp.zeros((NROWS - NN, HH), jnp.float32)


_tc_prep = pl.pallas_call(
    _tc_prep_body,
    out_shape=(
        jax.ShapeDtypeStruct((NN, HH), jnp.float32),
        jax.ShapeDtypeStruct((NROWS, HH), jnp.float32),
    ),
)


def _tc_layer_body(agg2_ref, x_ref, degT_ref, w_ref, b_ref, g_ref, be_ref,
                   xo_ref, xso_ref):
    p = agg2_ref[0, 0:NN, :] + agg2_ref[1, 0:NN, :]
    norm = lax.rsqrt(jnp.maximum(degT_ref[...], 1.0))
    a = p * norm[:, 1:2]
    z = lax.dot_general(a, w_ref[...], (((1,), (0,)), ((), ())),
                        preferred_element_type=jnp.float32) + b_ref[...]
    mean = jnp.mean(z, axis=0, keepdims=True)
    zc = z - mean
    var = jnp.mean(zc * zc, axis=0, keepdims=True)
    zn = g_ref[...] * zc / jnp.sqrt(var + 1e-5) + be_ref[...]
    xo = x_ref[...] + jnp.maximum(zn, 0.0)
    xo_ref[...] = xo
    xso_ref[0:NN, :] = xo * norm[:, 0:1]
    xso_ref[NN:NROWS, :] = jnp.zeros((NROWS - NN, HH), jnp.float32)


_tc_layer = pl.pallas_call(
    _tc_layer_body,
    out_shape=(
        jax.ShapeDtypeStruct((NN, HH), jnp.float32),
        jax.ShapeDtypeStruct((NROWS, HH), jnp.float32),
    ),
)


def _tc_readout_body(x_ref, w1_ref, b1_ref, w2_ref, b2_ref, w3_ref, b3_ref,
                     o_ref):
    hg = jnp.mean(x_ref[...], axis=0, keepdims=True)
    y = jnp.maximum(
        lax.dot_general(hg, w1_ref[...], (((1,), (0,)), ((), ())),
                        preferred_element_type=jnp.float32) + b1_ref[...], 0.0)
    y = jnp.maximum(
        lax.dot_general(y, w2_ref[...], (((1,), (0,)), ((), ())),
                        preferred_element_type=jnp.float32) + b2_ref[...], 0.0)
    o_ref[...] = lax.dot_general(y, w3_ref[...], (((1,), (0,)), ((), ())),
                                 preferred_element_type=jnp.float32) + b3_ref[...]


_tc_readout = pl.pallas_call(
    _tc_readout_body,
    out_shape=jax.ShapeDtypeStruct((1, 1), jnp.float32),
)


def kernel(edge_index, h, e, emb, Ws, bs, gammas, betas, W1, b1, W2, b2, W3, b3):
    del e
    src = edge_index[0]
    dst = edge_index[1]
    half = EE // NC

    def prep_idx(a):
        pad = jnp.full((EPT * NT - half,), DUMMY, jnp.int32)
        halves = [jnp.concatenate([a[i * half:(i + 1) * half], pad])
                  for i in range(NC)]
        return jnp.concatenate(halves).reshape(NC * NT, CPT, CH)

    srcp = prep_idx(src)
    dstp = prep_idx(dst)

    deg2 = _sc_degrees(srcp, dstp)
    degT = deg2[:, :NN].T
    x, xs = _tc_prep(h.reshape(NN, 1), emb, degT)
    for i in range(LL):
        agg2 = _sc_spmm(xs, srcp, dstp)
        x, xs = _tc_layer(agg2, x, degT, Ws[i], bs[i].reshape(1, HH),
                          gammas[i].reshape(1, HH), betas[i].reshape(1, HH))
    out = _tc_readout(x, W1, b1.reshape(1, HH // 2), W2, b2.reshape(1, HH // 4),
                      W3, b3.reshape(1, 1))
    return out.reshape(1)
